# Initial kernel scaffold; baseline (speedup 1.0000x reference)
import jax
import jax.numpy as jnp
from jax.experimental import pallas as pl


def kernel(x, W):
    raise NotImplementedError("write your pallas kernel here")


# speedup vs baseline: 1.3560x; 1.3560x over previous
import jax
import jax.numpy as jnp
from jax.experimental import pallas as pl

_ROWS = 512


def _router_kernel(x_ref, w_ref, probs_ref, idx_ref, wts_ref):
    x = x_ref[...]
    w = w_ref[...]
    logits = jax.lax.dot_general(
        x, w, (((1,), (1,)), ((), ())),
        preferred_element_type=jnp.float32,
        precision=jax.lax.Precision.DEFAULT,
    )
    m = jnp.max(logits, axis=-1, keepdims=True)
    e = jnp.exp(logits - m)
    s = jnp.sum(e, axis=-1, keepdims=True)
    probs = e / s
    probs_ref[...] = probs

    ncols = probs.shape[-1]
    iota = jax.lax.broadcasted_iota(jnp.int32, probs.shape, 1)
    p1 = jnp.max(probs, axis=-1, keepdims=True)
    idx1 = jnp.min(jnp.where(probs == p1, iota, ncols), axis=-1, keepdims=True)
    probs2 = jnp.where(iota == idx1, jnp.float32(-jnp.inf), probs)
    p2 = jnp.max(probs2, axis=-1, keepdims=True)
    idx2 = jnp.min(jnp.where(probs2 == p2, iota, ncols), axis=-1, keepdims=True)
    denom = p1 + p2 + jnp.float32(1e-9)
    idx_ref[...] = jnp.concatenate([idx1, idx2], axis=-1)
    wts_ref[...] = jnp.concatenate([p1 / denom, p2 / denom], axis=-1)


def kernel(x, W):
    B, T, D = x.shape
    N = B * T
    E = W.shape[0]
    x2 = x.reshape(N, D)
    R = _ROWS
    probs, idx, wts = pl.pallas_call(
        _router_kernel,
        grid=(N // R,),
        in_specs=[
            pl.BlockSpec((R, D), lambda i: (i, 0)),
            pl.BlockSpec((E, D), lambda i: (0, 0)),
        ],
        out_specs=[
            pl.BlockSpec((R, E), lambda i: (i, 0)),
            pl.BlockSpec((R, 2), lambda i: (i, 0)),
            pl.BlockSpec((R, 2), lambda i: (i, 0)),
        ],
        out_shape=[
            jax.ShapeDtypeStruct((N, E), jnp.float32),
            jax.ShapeDtypeStruct((N, 2), jnp.int32),
            jax.ShapeDtypeStruct((N, 2), jnp.float32),
        ],
    )(x2, W)
    return (probs, idx, wts)


# speedup vs baseline: 1.5641x; 1.1535x over previous
import jax
import jax.numpy as jnp
from jax.experimental import pallas as pl

_ROWS = 1024


def _router_kernel(x_ref, w_ref, probs_ref, idx_ref, wts_ref):
    x = x_ref[...]
    w = w_ref[...]
    logits = jax.lax.dot_general(
        x, w, (((1,), (1,)), ((), ())),
        preferred_element_type=jnp.float32,
        precision=jax.lax.Precision.DEFAULT,
    )
    m = jnp.max(logits, axis=-1, keepdims=True)
    e = jnp.exp(logits - m)
    s = jnp.sum(e, axis=-1, keepdims=True)
    probs = e / s
    probs_ref[...] = probs

    ncols = probs.shape[-1]
    iota = jax.lax.broadcasted_iota(jnp.int32, probs.shape, 1)
    p1 = jnp.max(probs, axis=-1, keepdims=True)
    idx1 = jnp.min(jnp.where(probs == p1, iota, ncols), axis=-1, keepdims=True)
    probs2 = jnp.where(iota == idx1, jnp.float32(-jnp.inf), probs)
    p2 = jnp.max(probs2, axis=-1, keepdims=True)
    idx2 = jnp.min(jnp.where(probs2 == p2, iota, ncols), axis=-1, keepdims=True)
    denom = p1 + p2 + jnp.float32(1e-9)
    idx_ref[...] = jnp.concatenate([idx1, idx2], axis=-1)
    wts_ref[...] = jnp.concatenate([p1 / denom, p2 / denom], axis=-1)


def kernel(x, W):
    B, T, D = x.shape
    N = B * T
    E = W.shape[0]
    x2 = x.reshape(N, D)
    R = _ROWS
    probs, idx, wts = pl.pallas_call(
        _router_kernel,
        grid=(N // R,),
        in_specs=[
            pl.BlockSpec((R, D), lambda i: (i, 0)),
            pl.BlockSpec((E, D), lambda i: (0, 0)),
        ],
        out_specs=[
            pl.BlockSpec((R, E), lambda i: (i, 0)),
            pl.BlockSpec((R, 2), lambda i: (i, 0)),
            pl.BlockSpec((R, 2), lambda i: (i, 0)),
        ],
        out_shape=[
            jax.ShapeDtypeStruct((N, E), jnp.float32),
            jax.ShapeDtypeStruct((N, 2), jnp.int32),
            jax.ShapeDtypeStruct((N, 2), jnp.float32),
        ],
    )(x2, W)
    return (probs, idx, wts)


# speedup vs baseline: 1.6211x; 1.0365x over previous
import jax
import jax.numpy as jnp
from jax.experimental import pallas as pl

_ROWS = 2048


def _router_kernel(x_ref, w_ref, probs_ref, idx_ref, wts_ref):
    x = x_ref[...]
    w = w_ref[...]
    logits = jax.lax.dot_general(
        x, w, (((1,), (1,)), ((), ())),
        preferred_element_type=jnp.float32,
        precision=jax.lax.Precision.DEFAULT,
    )
    m = jnp.max(logits, axis=-1, keepdims=True)
    e = jnp.exp(logits - m)
    s = jnp.sum(e, axis=-1, keepdims=True)
    probs = e / s
    probs_ref[...] = probs

    ncols = probs.shape[-1]
    iota = jax.lax.broadcasted_iota(jnp.int32, probs.shape, 1)
    p1 = jnp.max(probs, axis=-1, keepdims=True)
    idx1 = jnp.min(jnp.where(probs == p1, iota, ncols), axis=-1, keepdims=True)
    probs2 = jnp.where(iota == idx1, jnp.float32(-jnp.inf), probs)
    p2 = jnp.max(probs2, axis=-1, keepdims=True)
    idx2 = jnp.min(jnp.where(probs2 == p2, iota, ncols), axis=-1, keepdims=True)
    denom = p1 + p2 + jnp.float32(1e-9)
    idx_ref[...] = jnp.concatenate([idx1, idx2], axis=-1)
    wts_ref[...] = jnp.concatenate([p1 / denom, p2 / denom], axis=-1)


def kernel(x, W):
    B, T, D = x.shape
    N = B * T
    E = W.shape[0]
    x2 = x.reshape(N, D)
    R = _ROWS
    probs, idx, wts = pl.pallas_call(
        _router_kernel,
        grid=(N // R,),
        in_specs=[
            pl.BlockSpec((R, D), lambda i: (i, 0)),
            pl.BlockSpec((E, D), lambda i: (0, 0)),
        ],
        out_specs=[
            pl.BlockSpec((R, E), lambda i: (i, 0)),
            pl.BlockSpec((R, 2), lambda i: (i, 0)),
            pl.BlockSpec((R, 2), lambda i: (i, 0)),
        ],
        out_shape=[
            jax.ShapeDtypeStruct((N, E), jnp.float32),
            jax.ShapeDtypeStruct((N, 2), jnp.int32),
            jax.ShapeDtypeStruct((N, 2), jnp.float32),
        ],
    )(x2, W)
    return (probs, idx, wts)
